# Initial kernel scaffold; baseline (speedup 1.0000x reference)
import jax
import jax.numpy as jnp
from jax.experimental import pallas as pl


def kernel(x, H, W, ln1_g, ln1_b, ln2_g, ln2_b, W_rel, W_self, W_gate, b_gate, W_proj, b_proj, W_fc1, b_fc1, W_fc2, b_fc2):
    raise NotImplementedError("write your pallas kernel here")


# speedup vs baseline: 6.8861x; 6.8861x over previous
import jax
import jax.numpy as jnp
from jax.experimental import pallas as pl

_B, _L, _C = 32, 3136, 96
_DEPTH = 2
_R = 4
_FFN = _C * 4
_HH, _WW = 56, 56


def _ln(x, g, b):
    mu = jnp.mean(x, axis=-1, keepdims=True)
    var = jnp.mean((x - mu) ** 2, axis=-1, keepdims=True)
    return (x - mu) * jax.lax.rsqrt(var + 1e-5) * g + b


def _shift_down(a, k):
    return jnp.concatenate([jnp.zeros((k, a.shape[1]), a.dtype), a[:-k]], axis=0)


def _shift_up(a, k):
    return jnp.concatenate([a[k:], jnp.zeros((k, a.shape[1]), a.dtype)], axis=0)


def _stage_kernel(x_ref, ln1_g, ln1_b, ln2_g, ln2_b, W_rel, W_self, W_gate,
                  b_gate, W_proj, b_proj, W_fc1, b_fc1, W_fc2, b_fc2, o_ref):
    xb = x_ref[0]
    rows = jax.lax.broadcasted_iota(jnp.int32, (_L, 1), 0)
    col = rows % _WW
    m_not_first_col = (col != 0)
    m_not_last_col = (col != _WW - 1)

    for d in range(_DEPTH):
        h = _ln(xb, ln1_g[d], ln1_b[d])
        x5 = jnp.concatenate([
            h,
            jnp.where(m_not_first_col, _shift_down(h, 1), 0.0),
            jnp.where(m_not_last_col, _shift_up(h, 1), 0.0),
            _shift_down(h, _WW),
            _shift_up(h, _WW),
        ], axis=1)
        w5 = jnp.concatenate([W_self[d], W_rel[d, 0], W_rel[d, 1],
                              W_rel[d, 2], W_rel[d, 3]], axis=0)
        agg = jnp.dot(x5, w5, preferred_element_type=jnp.float32)
        gate = jax.nn.sigmoid(
            jnp.dot(h, W_gate[d], preferred_element_type=jnp.float32) + b_gate[d])
        conv = jax.nn.gelu(agg) * gate
        conv = jnp.dot(conv, W_proj[d], preferred_element_type=jnp.float32) + b_proj[d]
        xb = xb + conv
        h2 = _ln(xb, ln2_g[d], ln2_b[d])
        hid = jax.nn.gelu(
            jnp.dot(h2, W_fc1[d], preferred_element_type=jnp.float32) + b_fc1[d])
        xb = xb + jnp.dot(hid, W_fc2[d], preferred_element_type=jnp.float32) + b_fc2[d]

    o_ref[0] = xb


def kernel(x, H, W, ln1_g, ln1_b, ln2_g, ln2_b, W_rel, W_self, W_gate, b_gate,
           W_proj, b_proj, W_fc1, b_fc1, W_fc2, b_fc2):
    del H, W
    full = lambda shape: pl.BlockSpec(shape, lambda b: (0,) * len(shape))
    out = pl.pallas_call(
        _stage_kernel,
        grid=(_B,),
        in_specs=[
            pl.BlockSpec((1, _L, _C), lambda b: (b, 0, 0)),
            full((_DEPTH, _C)), full((_DEPTH, _C)),
            full((_DEPTH, _C)), full((_DEPTH, _C)),
            full((_DEPTH, _R, _C, _C)), full((_DEPTH, _C, _C)),
            full((_DEPTH, _C, _C)), full((_DEPTH, _C)),
            full((_DEPTH, _C, _C)), full((_DEPTH, _C)),
            full((_DEPTH, _C, _FFN)), full((_DEPTH, _FFN)),
            full((_DEPTH, _FFN, _C)), full((_DEPTH, _C)),
        ],
        out_specs=pl.BlockSpec((1, _L, _C), lambda b: (b, 0, 0)),
        out_shape=jax.ShapeDtypeStruct((_B, _L, _C), jnp.float32),
    )(x, ln1_g, ln1_b, ln2_g, ln2_b, W_rel, W_self, W_gate, b_gate,
      W_proj, b_proj, W_fc1, b_fc1, W_fc2, b_fc2)
    return out
